# baseline (speedup 1.0000x reference)
import functools
import jax
import jax.numpy as jnp
from jax import lax
from jax.experimental import pallas as pl
from jax.experimental.pallas import tpu as pltpu
from jax.experimental.pallas import tpu_sc as plsc

_NUM_CORES = 2
_NUM_SUBCORES = 16
_NUM_WORKERS = _NUM_CORES * _NUM_SUBCORES
_LANES = 16
_GATHER_CHUNK = 128


@functools.lru_cache(maxsize=None)
def _build(B, D, dtype_name):
    dtype = jnp.dtype(dtype_name)
    b_per_w = B // _NUM_WORKERS
    n_chunks = b_per_w // _GATHER_CHUNK
    mesh = plsc.VectorSubcoreMesh(
        core_axis_name="c", subcore_axis_name="s",
        num_cores=_NUM_CORES, num_subcores=_NUM_SUBCORES,
    )

    @functools.partial(
        pl.kernel,
        out_type=jax.ShapeDtypeStruct((B, D), dtype),
        mesh=mesh,
        scratch_types=[
            pltpu.VMEM((b_per_w,), jnp.int32),
            pltpu.VMEM((b_per_w,), jnp.int32),
            pltpu.VMEM((b_per_w,), jnp.int32),
            pltpu.VMEM((b_per_w, D), dtype),
            pltpu.VMEM((b_per_w, D), dtype),
            pltpu.VMEM((b_per_w, D), dtype),
            pltpu.SemaphoreType.DMA,
        ],
        compiler_params=pltpu.CompilerParams(use_tc_tiling_on_sc=False),
    )
    def dist_mult(s_hbm, r_hbm, o_hbm, emb_e_hbm, emb_rel_hbm, out_hbm,
                  idx_s, idx_r, idx_o, rows_s, rows_r, rows_o, sem):
        wid = lax.axis_index("s") * _NUM_CORES + lax.axis_index("c")
        base = wid * b_per_w

        pltpu.sync_copy(s_hbm.at[pl.ds(base, b_per_w)], idx_s)
        pltpu.sync_copy(r_hbm.at[pl.ds(base, b_per_w)], idx_r)
        pltpu.sync_copy(o_hbm.at[pl.ds(base, b_per_w)], idx_o)

        copies = []
        for j in range(n_chunks):
            sl = pl.ds(j * _GATHER_CHUNK, _GATHER_CHUNK)
            copies.append(pltpu.async_copy(
                emb_e_hbm.at[idx_s.at[sl]], rows_s.at[sl], sem))
            copies.append(pltpu.async_copy(
                emb_rel_hbm.at[idx_r.at[sl]], rows_r.at[sl], sem))
            copies.append(pltpu.async_copy(
                emb_e_hbm.at[idx_o.at[sl]], rows_o.at[sl], sem))
        for c in copies:
            c.wait()

        def body(i, _):
            for col in range(D // _LANES):
                cs = pl.ds(col * _LANES, _LANES)
                rows_s[i, cs] = rows_s[i, cs] * rows_r[i, cs] * rows_o[i, cs]
            return 0
        lax.fori_loop(0, b_per_w, body, 0)

        pltpu.sync_copy(rows_s, out_hbm.at[pl.ds(base, b_per_w)])

    return dist_mult


def kernel(emb_e, emb_rel, triplets):
    B, D = triplets.shape[0], emb_e.shape[1]
    s_idx = triplets[:, 0]
    r_idx = triplets[:, 1]
    o_idx = triplets[:, 2]
    fn = _build(B, D, emb_e.dtype.name)
    return fn(s_idx, r_idx, o_idx, emb_e, emb_rel)
